# baseline (speedup 1.0000x reference)
import functools
import math

import jax
import jax.numpy as jnp
from jax import lax
from jax.experimental import pallas as pl
from jax.experimental.pallas import tpu as pltpu
from jax.experimental.pallas import tpu_sc as plsc

D_MODEL = 128
SCALE = math.sqrt(float(D_MODEL))

_info = plsc.get_sparse_core_info()
_NC, _NS, _L = _info.num_cores, _info.num_subcores, _info.num_lanes
_NW = _NC * _NS

CHUNK = 128


def _make_kernel(B):
    assert B % (_NW * CHUNK) == 0
    b_per_w = B // _NW
    n_chunks = b_per_w // CHUNK
    mesh = plsc.VectorSubcoreMesh(core_axis_name="c", subcore_axis_name="s")

    @functools.partial(
        pl.kernel,
        mesh=mesh,
        out_type=jax.ShapeDtypeStruct((B, D_MODEL), jnp.float32),
        scratch_types=[
            pltpu.VMEM((CHUNK,), jnp.int32),
            pltpu.VMEM((CHUNK, D_MODEL), jnp.float32),
            pltpu.SemaphoreType.DMA,
        ],
    )
    def k(idx_hbm, table_hbm, out_hbm, idx_v, rows_v, sem):
        wid = lax.axis_index("s") * _NC + lax.axis_index("c")
        w_base = wid * b_per_w

        def body(t, carry):
            base = w_base + t * CHUNK
            pltpu.sync_copy(idx_hbm.at[pl.ds(base, CHUNK)], idx_v)
            pltpu.async_copy(table_hbm.at[idx_v], rows_v, sem).wait()

            def scale_row(i, c):
                for j in range(D_MODEL // _L):
                    sl = pl.ds(j * _L, _L)
                    rows_v[i, sl] = rows_v[i, sl] * SCALE
                return c

            lax.fori_loop(0, CHUNK, scale_row, 0)
            pltpu.sync_copy(rows_v, out_hbm.at[pl.ds(base, CHUNK)])
            return carry

        lax.fori_loop(0, n_chunks, body, 0)

    return k


def kernel(x, table):
    orig_shape = x.shape
    flat = jnp.reshape(x, (-1,)).astype(jnp.int32)
    out = _make_kernel(flat.shape[0])(flat, table)
    return jnp.reshape(out, orig_shape + (D_MODEL,))
